# baseline (speedup 1.0000x reference)
import jax
import jax.numpy as jnp
from jax.experimental import pallas as pl
from jax.experimental.pallas import tpu as pltpu

B, S, D = 4, 2048, 2048
NUM_ROUTED, TOP_K, RANK = 8, 2, 64
N_TOK = B * S

TM = 512
W1_COLS = D + 128 + 64 + NUM_ROUTED * RANK + 64
H_OFF = D + 128
H_COLS = 64 + NUM_ROUTED * RANK + 64


def _fused_kernel(x_ref, w1_ref, w2_ref, exp_ref, b_ref, br_ref, o_ref):
    xb = x_ref[...].astype(jnp.bfloat16)
    h = jax.lax.dot_general(
        xb, w1_ref[...], (((1,), (0,)), ((), ())),
        preferred_element_type=jnp.float32)

    logits = h[:, D:D + NUM_ROUTED] + br_ref[...]
    m = jnp.max(logits, axis=1, keepdims=True)
    e = jnp.exp(logits - m)
    p = e / jnp.sum(e, axis=1, keepdims=True)

    idx = jax.lax.broadcasted_iota(jnp.int32, p.shape, 1)
    m1 = jnp.max(p, axis=1, keepdims=True)
    i1 = jnp.min(jnp.where(p == m1, idx, NUM_ROUTED), axis=1, keepdims=True)
    mask1 = idx == i1
    p2 = jnp.where(mask1, -1.0, p)
    m2 = jnp.max(p2, axis=1, keepdims=True)
    i2 = jnp.min(jnp.where(p2 == m2, idx, NUM_ROUTED), axis=1, keepdims=True)
    s = jnp.where(mask1 | (idx == i2), p, 0.0)

    scale = jax.lax.dot_general(
        s.astype(jnp.bfloat16), exp_ref[...], (((1,), (0,)), ((), ())),
        preferred_element_type=jnp.float32)
    cols = jax.lax.broadcasted_iota(jnp.int32, (TM, H_COLS), 1)
    scale = jnp.where(cols < RANK, 1.0, scale)

    hs = (h[:, H_OFF:H_OFF + H_COLS] * scale).astype(jnp.bfloat16)
    out = h[:, :D] + jax.lax.dot_general(
        hs, w2_ref[...], (((1,), (0,)), ((), ())),
        preferred_element_type=jnp.float32)
    o_ref[...] = out + b_ref[...]


@jax.jit
def kernel(x, W_orig, b_orig, Ws1, Ws2, Wr1, Wr2, W_router, b_router):
    xf = x.reshape(N_TOK, D)

    w1 = jnp.concatenate([
        W_orig.astype(jnp.bfloat16).T,
        W_router.astype(jnp.bfloat16).T,
        jnp.zeros((D, 120), jnp.bfloat16),
        Ws1[0].astype(jnp.bfloat16).T,
        Wr1.astype(jnp.bfloat16).transpose(2, 0, 1).reshape(D, NUM_ROUTED * RANK),
        jnp.zeros((D, 64), jnp.bfloat16),
    ], axis=1)

    w2 = jnp.concatenate([
        Ws2[0].astype(jnp.bfloat16).T,
        Wr2.astype(jnp.bfloat16).transpose(0, 2, 1).reshape(NUM_ROUTED * RANK, D),
        jnp.zeros((64, D), jnp.bfloat16),
    ], axis=0)

    rows = jnp.arange(NUM_ROUTED)[:, None]
    cc = jnp.arange(H_COLS)[None, :]
    expand = ((cc >= RANK + rows * RANK) & (cc < RANK + (rows + 1) * RANK)
              ).astype(jnp.bfloat16)

    bias = b_orig[None, :]
    rbias = b_router[None, :]

    out = pl.pallas_call(
        _fused_kernel,
        grid=(N_TOK // TM,),
        in_specs=[
            pl.BlockSpec((TM, D), lambda i: (i, 0)),
            pl.BlockSpec((D, W1_COLS), lambda i: (0, 0)),
            pl.BlockSpec((H_COLS, D), lambda i: (0, 0)),
            pl.BlockSpec((NUM_ROUTED, H_COLS), lambda i: (0, 0)),
            pl.BlockSpec((1, D), lambda i: (0, 0)),
            pl.BlockSpec((1, NUM_ROUTED), lambda i: (0, 0)),
        ],
        out_specs=pl.BlockSpec((TM, D), lambda i: (i, 0)),
        out_shape=jax.ShapeDtypeStruct((N_TOK, D), jnp.float32),
        compiler_params=pltpu.CompilerParams(
            dimension_semantics=("arbitrary",),
        ),
    )(xf, w1, w2, expand, bias, rbias)

    return out.reshape(B, S, D)


# speedup vs baseline: 1.0824x; 1.0824x over previous
import jax
import jax.numpy as jnp
from jax.experimental import pallas as pl
from jax.experimental.pallas import tpu as pltpu

B, S, D = 4, 2048, 2048
NUM_ROUTED, TOP_K, RANK = 8, 2, 64
N_TOK = B * S

TM = 512
W1_COLS = D + 128 + 64 + NUM_ROUTED * RANK + 64
H_OFF = D + 128
H_COLS = 64 + NUM_ROUTED * RANK + 64


def _fused_kernel(x_ref, w1_ref, w2_ref, exp_ref, b_ref, br_ref, o_ref):
    xb = x_ref[...].astype(jnp.bfloat16)
    h = jax.lax.dot_general(
        xb, w1_ref[...], (((1,), (1,)), ((), ())),
        preferred_element_type=jnp.float32)

    logits = h[:, D:D + NUM_ROUTED] + br_ref[...]
    m = jnp.max(logits, axis=1, keepdims=True)
    e = jnp.exp(logits - m)
    p = e / jnp.sum(e, axis=1, keepdims=True)

    idx = jax.lax.broadcasted_iota(jnp.int32, p.shape, 1)
    m1 = jnp.max(p, axis=1, keepdims=True)
    i1 = jnp.min(jnp.where(p == m1, idx, NUM_ROUTED), axis=1, keepdims=True)
    mask1 = idx == i1
    p2 = jnp.where(mask1, -1.0, p)
    m2 = jnp.max(p2, axis=1, keepdims=True)
    i2 = jnp.min(jnp.where(p2 == m2, idx, NUM_ROUTED), axis=1, keepdims=True)
    s = jnp.where(mask1 | (idx == i2), p, 0.0)

    scale = jax.lax.dot_general(
        s.astype(jnp.bfloat16), exp_ref[...], (((1,), (0,)), ((), ())),
        preferred_element_type=jnp.float32)
    cols = jax.lax.broadcasted_iota(jnp.int32, (TM, H_COLS), 1)
    scale = jnp.where(cols < RANK, 1.0, scale)

    hs = (h[:, H_OFF:H_OFF + H_COLS] * scale).astype(jnp.bfloat16)
    out = h[:, :D] + jax.lax.dot_general(
        hs, w2_ref[...], (((1,), (1,)), ((), ())),
        preferred_element_type=jnp.float32)
    o_ref[...] = out + b_ref[...]


@jax.jit
def kernel(x, W_orig, b_orig, Ws1, Ws2, Wr1, Wr2, W_router, b_router):
    xf = x.reshape(N_TOK, D)

    w1 = jnp.concatenate([
        W_orig.astype(jnp.bfloat16),
        W_router.astype(jnp.bfloat16),
        jnp.zeros((120, D), jnp.bfloat16),
        Ws1[0].astype(jnp.bfloat16),
        Wr1.reshape(NUM_ROUTED * RANK, D).astype(jnp.bfloat16),
        jnp.zeros((64, D), jnp.bfloat16),
    ], axis=0)

    w2 = jnp.concatenate([
        Ws2[0].astype(jnp.bfloat16),
        Wr2.transpose(1, 0, 2).reshape(D, NUM_ROUTED * RANK).astype(jnp.bfloat16),
        jnp.zeros((D, 64), jnp.bfloat16),
    ], axis=1)

    rows = jnp.arange(NUM_ROUTED)[:, None]
    cc = jnp.arange(H_COLS)[None, :]
    expand = ((cc >= RANK + rows * RANK) & (cc < RANK + (rows + 1) * RANK)
              ).astype(jnp.bfloat16)

    bias = b_orig[None, :]
    rbias = b_router[None, :]

    out = pl.pallas_call(
        _fused_kernel,
        grid=(N_TOK // TM,),
        in_specs=[
            pl.BlockSpec((TM, D), lambda i: (i, 0)),
            pl.BlockSpec((W1_COLS, D), lambda i: (0, 0)),
            pl.BlockSpec((D, H_COLS), lambda i: (0, 0)),
            pl.BlockSpec((NUM_ROUTED, H_COLS), lambda i: (0, 0)),
            pl.BlockSpec((1, D), lambda i: (0, 0)),
            pl.BlockSpec((1, NUM_ROUTED), lambda i: (0, 0)),
        ],
        out_specs=pl.BlockSpec((TM, D), lambda i: (i, 0)),
        out_shape=jax.ShapeDtypeStruct((N_TOK, D), jnp.float32),
        compiler_params=pltpu.CompilerParams(
            dimension_semantics=("arbitrary",),
        ),
    )(xf, w1, w2, expand, bias, rbias)

    return out.reshape(B, S, D)
